# baseline (speedup 1.0000x reference)
import functools

import jax
import jax.numpy as jnp
from jax import lax
from jax.experimental import pallas as pl
from jax.experimental.pallas import tpu as pltpu

_LANES = 128
_ALIGN = 1024
_ROWS_PER_STEP = 4
_LAG_STEPS = 2


def _tc_body(q_len, v_len, dim, max_pos, region_words, n_steps,
             big_ref, out_ref, sem):
  i = pl.program_id(0)
  row_rows = v_len * dim // _LANES

  def row_copy(step, t):
    r = step * _ROWS_PER_STEP + t
    q_row = lax.rem(r, q_len)
    s = (max_pos - q_row) * dim
    j = lax.rem(lax.div(s, 64), 16)
    pad = lax.rem((16 - j) * 64, _ALIGN)
    src_row = lax.div(j * region_words + pad + s, _LANES)
    return pltpu.make_async_copy(
        big_ref.at[pl.ds(src_row, row_rows), :],
        out_ref.at[pl.ds(r * row_rows, row_rows), :],
        sem,
    )

  @pl.when(i < n_steps)
  def _start():
    for t in range(_ROWS_PER_STEP):
      row_copy(i, t).start()

  @pl.when(i >= _LAG_STEPS)
  def _wait():
    for t in range(_ROWS_PER_STEP):
      row_copy(i - _LAG_STEPS, t).wait()


def kernel(q, v, embeddings):
  batch, q_len = q.shape[0], q.shape[1]
  v_len = v.shape[1]
  table_rows, dim = embeddings.shape
  max_pos = (table_rows - 1) // 2

  table_words = table_rows * dim
  region_words = -(-(960 + table_words) // _ALIGN) * _ALIGN

  flat = embeddings.reshape(-1)
  big = jnp.zeros((16 * region_words,), jnp.float32)
  for j in range(16):
    pad = (16 - j) * 64 % _ALIGN
    big = lax.dynamic_update_slice(big, flat, (j * region_words + pad,))
  big2d = big.reshape(-1, _LANES)
  n_big_rows = big2d.shape[0]

  n_rows = batch * q_len
  n_steps = n_rows // _ROWS_PER_STEP
  row_rows = v_len * dim // _LANES
  body = functools.partial(
      _tc_body, q_len, v_len, dim, max_pos, region_words, n_steps)
  out = pl.pallas_call(
      body,
      grid=(n_steps + _LAG_STEPS,),
      in_specs=[pl.BlockSpec((n_big_rows, _LANES), lambda i: (0, 0))],
      out_specs=pl.BlockSpec(memory_space=pl.ANY),
      out_shape=jax.ShapeDtypeStruct((n_rows * row_rows, _LANES),
                                     jnp.float32),
      scratch_shapes=[pltpu.SemaphoreType.DMA],
  )(big2d)
  return out.reshape(batch, q_len, v_len, dim)


# speedup vs baseline: 1.2824x; 1.2824x over previous
import functools

import jax
import jax.numpy as jnp
from jax import lax
from jax.experimental import pallas as pl

_LANES = 128
_ALIGN = 1024
_QB = 8


def _tc_body(q_len, v_len, dim, max_pos, region_words, big_ref, out_ref):
  j = pl.program_id(1)
  row_rows = v_len * dim // _LANES
  for r in range(_QB):
    q_row = j * _QB + r
    s = (max_pos - q_row) * dim
    jj = lax.rem(lax.div(s, 64), 16)
    pad = lax.rem((16 - jj) * 64, _ALIGN)
    src_row = lax.div(jj * region_words + pad + s, _LANES)
    out_ref[0, r] = big_ref[pl.ds(src_row, row_rows), :]


def kernel(q, v, embeddings):
  batch, q_len = q.shape[0], q.shape[1]
  v_len = v.shape[1]
  table_rows, dim = embeddings.shape
  max_pos = (table_rows - 1) // 2

  table_words = table_rows * dim
  region_words = -(-(960 + table_words) // _ALIGN) * _ALIGN

  flat = embeddings.reshape(-1)
  big = jnp.zeros((16 * region_words,), jnp.float32)
  for j in range(16):
    pad = (16 - j) * 64 % _ALIGN
    big = lax.dynamic_update_slice(big, flat, (j * region_words + pad,))
  big2d = big.reshape(-1, _LANES)
  n_big_rows = big2d.shape[0]

  row_rows = v_len * dim // _LANES
  grid = (batch, q_len // _QB)
  body = functools.partial(
      _tc_body, q_len, v_len, dim, max_pos, region_words)
  out = pl.pallas_call(
      body,
      grid=grid,
      in_specs=[pl.BlockSpec((n_big_rows, _LANES), lambda b, j: (0, 0))],
      out_specs=pl.BlockSpec((1, _QB, row_rows, _LANES),
                             lambda b, j: (b, j, 0, 0)),
      out_shape=jax.ShapeDtypeStruct((batch, q_len, row_rows, _LANES),
                                     jnp.float32),
  )(big2d)
  return out.reshape(batch, q_len, v_len, dim)
